# baseline (speedup 1.0000x reference)
import jax
import jax.numpy as jnp
from jax.experimental import pallas as pl
from jax.experimental.pallas import tpu as pltpu

_ROW_BLK = 512
_LANE = 128
_ROWG = 64
_BIG = 2**30


def _nn_kernel(x_ref, ct_ref, out_ref, mm_ref):
    x = x_ref[...]
    ct = ct_ref[...]
    b = x.shape[0]
    k = ct.shape[1]
    nchunk = k // _LANE
    mm_ref[...] = jax.lax.dot_general(
        x * (-2.0), ct, (((1,), (0,)), ((), ())),
        preferred_element_type=jnp.float32,
    )
    x_sq = jnp.sum(x * x, axis=1)[:, None]
    c_sq = jax.lax.dot_general(
        jnp.ones((1, x.shape[1]), jnp.float32), ct * ct,
        (((1,), (0,)), ((), ())),
        preferred_element_type=jnp.float32,
    )

    lane = jax.lax.broadcasted_iota(jnp.int32, (_ROWG, _LANE), 1)
    for g in range(b // _ROWG):
        ro = g * _ROWG
        xs = x_sq[ro:ro + _ROWG]
        m = (xs + mm_ref[ro:ro + _ROWG, 0:_LANE]) + c_sq[:, 0:_LANE]
        ridx = jnp.zeros((_ROWG, _LANE), jnp.int32)
        for r in range(1, nchunk):
            lo = r * _LANE
            v = (xs + mm_ref[ro:ro + _ROWG, lo:lo + _LANE]) + c_sq[:, lo:lo + _LANE]
            keep = v < m
            ridx = jnp.where(keep, r, ridx)
            m = jnp.minimum(v, m)

        j = ridx * _LANE + lane
        mfin = jnp.min(m, axis=1, keepdims=True)
        idx = jnp.min(jnp.where(m == mfin, j, _BIG), axis=1)
        out_ref[ro:ro + _ROWG, :] = idx[:, None]


def kernel(x, centroids):
    lead = x.shape[:-1]
    fdim = x.shape[-1]
    flat = x.reshape(-1, fdim)
    n = flat.shape[0]
    k = centroids.shape[0]
    ct = centroids.T
    blk = _ROW_BLK
    assert n % blk == 0, (n, blk)
    nblk = n // blk
    out = pl.pallas_call(
        _nn_kernel,
        grid=(nblk,),
        in_specs=[
            pl.BlockSpec((blk, fdim), lambda i: (i, 0)),
            pl.BlockSpec(ct.shape, lambda i: (0, 0)),
        ],
        out_specs=pl.BlockSpec((blk, 1), lambda i: (i, 0)),
        out_shape=jax.ShapeDtypeStruct((n, 1), jnp.int32),
        scratch_shapes=[pltpu.VMEM((blk, k), jnp.float32)],
    )(flat, ct)
    return out.reshape(lead)


# speedup vs baseline: 1.4893x; 1.4893x over previous
import jax
import jax.numpy as jnp
from jax.experimental import pallas as pl


_ROW_BLK = 1024


def _nn_kernel(x_ref, ct_ref, out_ref):
    x = x_ref[...]
    ct = ct_ref[...]
    mm = jax.lax.dot_general(
        x * (-2.0), ct, (((1,), (0,)), ((), ())),
        preferred_element_type=jnp.float32,
    )
    x_sq = jnp.sum(x * x, axis=1)[:, None]
    c_sq = jnp.sum(ct * ct, axis=0)[None, :]
    d = (x_sq + mm) + c_sq
    idx = jnp.argmin(d, axis=1).astype(jnp.int32)
    out_ref[...] = idx[:, None]


def kernel(x, centroids):
    lead = x.shape[:-1]
    fdim = x.shape[-1]
    flat = x.reshape(-1, fdim)
    n = flat.shape[0]
    ct = centroids.T
    blk = _ROW_BLK
    assert n % blk == 0, (n, blk)
    nblk = n // blk
    out = pl.pallas_call(
        _nn_kernel,
        grid=(nblk,),
        in_specs=[
            pl.BlockSpec((blk, fdim), lambda i: (i, 0)),
            pl.BlockSpec(ct.shape, lambda i: (0, 0)),
        ],
        out_specs=pl.BlockSpec((blk, 1), lambda i: (i, 0)),
        out_shape=jax.ShapeDtypeStruct((n, 1), jnp.int32),
    )(flat, ct)
    return out.reshape(lead)


# speedup vs baseline: 2.4598x; 1.6517x over previous
import jax
import jax.numpy as jnp
from jax.experimental import pallas as pl


_BATCH_BLK = 8


def _nn_kernel(xt_ref, ct_ref, out_ref):
    xt = xt_ref[...]
    ct = ct_ref[...]
    bb, dd, tt = xt.shape
    x = jnp.swapaxes(xt, 1, 2).reshape(bb * tt, dd)
    mm = jax.lax.dot_general(
        x * (-2.0), ct, (((1,), (0,)), ((), ())),
        preferred_element_type=jnp.float32,
    )
    x_sq = jnp.sum(x * x, axis=1)[:, None]
    c_sq = jnp.sum(ct * ct, axis=0)[None, :]
    d = (x_sq + mm) + c_sq
    idx = jnp.argmin(d, axis=1).astype(jnp.int32)
    out_ref[...] = idx.reshape(bb, tt)


def kernel(x, centroids):
    batch, tokens, fdim = x.shape
    xt = jnp.swapaxes(x, 1, 2)
    ct = centroids.T
    bb = _BATCH_BLK
    assert batch % bb == 0, (batch, bb)
    nblk = batch // bb
    out = pl.pallas_call(
        _nn_kernel,
        grid=(nblk,),
        in_specs=[
            pl.BlockSpec((bb, fdim, tokens), lambda i: (i, 0, 0)),
            pl.BlockSpec(ct.shape, lambda i: (0, 0)),
        ],
        out_specs=pl.BlockSpec((bb, tokens), lambda i: (i, 0)),
        out_shape=jax.ShapeDtypeStruct((batch, tokens), jnp.int32),
    )(xt, ct)
    return out


# speedup vs baseline: 2.4944x; 1.0141x over previous
import jax
import jax.numpy as jnp
from jax.experimental import pallas as pl


_BATCH_BLK = 16


def _nn_kernel(xt_ref, ct_ref, out_ref):
    xt = xt_ref[...]
    ct = ct_ref[...]
    bb, dd, tt = xt.shape
    x = jnp.swapaxes(xt, 1, 2).reshape(bb * tt, dd)
    mm = jax.lax.dot_general(
        x * (-2.0), ct, (((1,), (0,)), ((), ())),
        preferred_element_type=jnp.float32,
    )
    x_sq = jnp.sum(x * x, axis=1)[:, None]
    c_sq = jnp.sum(ct * ct, axis=0)[None, :]
    d = (x_sq + mm) + c_sq
    idx = jnp.argmin(d, axis=1).astype(jnp.int32)
    out_ref[...] = idx.reshape(bb, tt)


def kernel(x, centroids):
    batch, tokens, fdim = x.shape
    xt = jnp.swapaxes(x, 1, 2)
    ct = centroids.T
    bb = _BATCH_BLK
    assert batch % bb == 0, (batch, bb)
    nblk = batch // bb
    out = pl.pallas_call(
        _nn_kernel,
        grid=(nblk,),
        in_specs=[
            pl.BlockSpec((bb, fdim, tokens), lambda i: (i, 0, 0)),
            pl.BlockSpec(ct.shape, lambda i: (0, 0)),
        ],
        out_specs=pl.BlockSpec((bb, tokens), lambda i: (i, 0)),
        out_shape=jax.ShapeDtypeStruct((batch, tokens), jnp.int32),
    )(xt, ct)
    return out


# speedup vs baseline: 2.8758x; 1.1529x over previous
import jax
import jax.numpy as jnp
from jax.experimental import pallas as pl


_BATCH_BLK = 16


def _nn_kernel(xt_ref, ct_ref, out_ref):
    xt = xt_ref[...]
    ct = ct_ref[...]
    bb, dd, tt = xt.shape
    x = jnp.swapaxes(xt, 1, 2).reshape(bb * tt, dd)
    mm = jax.lax.dot_general(
        x * (-2.0), ct, (((1,), (0,)), ((), ())),
        preferred_element_type=jnp.float32,
    )
    c_sq = jnp.sum(ct * ct, axis=0)[None, :]
    d = mm + c_sq
    idx = jnp.argmin(d, axis=1).astype(jnp.int32)
    out_ref[...] = idx.reshape(bb, tt)


def kernel(x, centroids):
    batch, tokens, fdim = x.shape
    xt = jnp.swapaxes(x, 1, 2)
    ct = centroids.T
    bb = _BATCH_BLK
    assert batch % bb == 0, (batch, bb)
    nblk = batch // bb
    out = pl.pallas_call(
        _nn_kernel,
        grid=(nblk,),
        in_specs=[
            pl.BlockSpec((bb, fdim, tokens), lambda i: (i, 0, 0)),
            pl.BlockSpec(ct.shape, lambda i: (0, 0)),
        ],
        out_specs=pl.BlockSpec((bb, tokens), lambda i: (i, 0)),
        out_shape=jax.ShapeDtypeStruct((batch, tokens), jnp.int32),
    )(xt, ct)
    return out


# speedup vs baseline: 2.9769x; 1.0351x over previous
import jax
import jax.numpy as jnp
from jax.experimental import pallas as pl


_BATCH_BLK = 16


def _nn_kernel(xt_ref, ct_ref, out_ref):
    xt = xt_ref[...]
    ct = ct_ref[...]
    bb, dd, tt = xt.shape
    xm2 = xt * (-2.0)
    mm = jnp.concatenate(
        [jax.lax.dot_general(
            xm2[i], ct, (((0,), (0,)), ((), ())),
            preferred_element_type=jnp.float32,
        ) for i in range(bb)], axis=0)
    c_sq = jnp.sum(ct * ct, axis=0)[None, :]
    d = mm + c_sq
    idx = jnp.argmin(d, axis=1).astype(jnp.int32)
    out_ref[...] = idx.reshape(bb, tt)


def kernel(x, centroids):
    batch, tokens, fdim = x.shape
    xt = jnp.swapaxes(x, 1, 2)
    ct = centroids.T
    bb = _BATCH_BLK
    assert batch % bb == 0, (batch, bb)
    nblk = batch // bb
    out = pl.pallas_call(
        _nn_kernel,
        grid=(nblk,),
        in_specs=[
            pl.BlockSpec((bb, fdim, tokens), lambda i: (i, 0, 0)),
            pl.BlockSpec(ct.shape, lambda i: (0, 0)),
        ],
        out_specs=pl.BlockSpec((bb, tokens), lambda i: (i, 0)),
        out_shape=jax.ShapeDtypeStruct((batch, tokens), jnp.int32),
    )(xt, ct)
    return out
